# baseline (device time: 74392 ns/iter reference)
import jax
import jax.numpy as jnp
from jax import lax
from jax.experimental import pallas as pl
from jax.experimental.pallas import tpu as pltpu

N_DEV = 32
N_TOK = 2048
D_IN = 512
D_OUT = 1024
E_LOC = 4
E_TOT = 128
CHUNK = N_TOK // N_DEV
CAP = 24
PK = N_DEV * CAP


def kernel(x, router_W, route_idx, expert_W):
    def body(x_ref, rw_ref, idx_ref, ew_ref, out_ref,
             partial_ref, pack_ref, recv_ref, send_sems, recv_sems):
        k = lax.axis_index("i")

        barrier = pltpu.get_barrier_semaphore()
        for d in range(N_DEV):
            @pl.when(d != k)
            def _():
                pl.semaphore_signal(barrier, inc=1, device_id=(d,),
                                    device_id_type=pl.DeviceIdType.MESH)
        pl.semaphore_wait(barrier, N_DEV - 1)

        recv_ref[k] = jnp.zeros((CAP, D_OUT), jnp.bfloat16)

        xv = x_ref[:, :]
        scores = jnp.dot(xv, rw_ref[:, :],
                         preferred_element_type=jnp.float32)
        m = jnp.max(scores, axis=-1, keepdims=True)
        p = jnp.exp(scores - m)
        probs = p / jnp.sum(p, axis=-1, keepdims=True)
        idx0 = idx_ref[:, 0:1]
        idx1 = idx_ref[:, 1:2]
        eids = lax.broadcasted_iota(jnp.int32, (N_TOK, E_TOT), 1)
        p0 = jnp.sum(jnp.where(eids == idx0, probs, 0.0), axis=-1,
                     keepdims=True)
        p1 = jnp.sum(jnp.where(eids == idx1, probs, 0.0), axis=-1,
                     keepdims=True)
        gsum = p0 + p1

        acc = None
        for j in range(E_LOC):
            e_id = E_LOC * k + j
            sel = jnp.sum(jnp.where(eids == e_id, probs, 0.0), axis=-1,
                          keepdims=True)
            hit = jnp.logical_or(idx0 == e_id, idx1 == e_id)
            w_j = jnp.where(hit, sel / gsum, 0.0)
            pj = jnp.dot(xv * w_j, ew_ref[j],
                         preferred_element_type=jnp.float32)
            acc = pj if acc is None else acc + pj
        partial_ref[:, :] = acc

        src0 = lax.div(idx0, E_LOC)
        src1 = lax.div(idx1, E_LOC)
        mymatch = jnp.logical_or(src0 == k, src1 == k)
        m_colf = mymatch.astype(jnp.float32)
        bd_i = lax.broadcasted_iota(jnp.int32, (N_TOK, N_TOK), 0)
        bd_j = lax.broadcasted_iota(jnp.int32, (N_TOK, N_TOK), 1)
        bd_excl = jnp.where(
            jnp.logical_and(lax.div(bd_i, CHUNK) == lax.div(bd_j, CHUNK),
                            bd_j < bd_i),
            1.0, 0.0)
        pos_col = jnp.dot(bd_excl, m_colf,
                          preferred_element_type=jnp.float32)
        trow = lax.broadcasted_iota(jnp.int32, (N_TOK, PK), 0)
        ccol = lax.broadcasted_iota(jnp.int32, (N_TOK, PK), 1)
        p_bigT = jnp.where(
            jnp.logical_and(
                jnp.logical_and(pos_col == (ccol % CAP).astype(jnp.float32),
                                m_colf > 0.5),
                lax.div(trow, CHUNK) == lax.div(ccol, CAP)),
            1.0, 0.0)
        packed = lax.dot_general(p_bigT, partial_ref[:, :],
                                 (((0,), (0,)), ((), ())),
                                 preferred_element_type=jnp.float32)
        pack_ref[:, :, :] = packed.astype(jnp.bfloat16).reshape(
            N_DEV, CAP, D_OUT)

        def a2a_copy(d):
            return pltpu.make_async_remote_copy(
                src_ref=pack_ref.at[d],
                dst_ref=recv_ref.at[k],
                send_sem=send_sems.at[d],
                recv_sem=recv_sems.at[k],
                device_id=(d,),
                device_id_type=pl.DeviceIdType.MESH,
            )

        for d in range(N_DEV):
            @pl.when(d != k)
            def _():
                a2a_copy(d).start()

        my0 = idx_ref[pl.ds(k * CHUNK, CHUNK), 0:1]
        my1 = idx_ref[pl.ds(k * CHUNK, CHUNK), 1:2]
        s_exp = lax.div(lax.broadcasted_iota(jnp.int32, (CHUNK, PK), 1), CAP)
        cap_exp = lax.broadcasted_iota(jnp.int32, (CHUNK, PK), 1) % CAP
        m_exp = jnp.logical_or(lax.div(my0, E_LOC) == s_exp,
                               lax.div(my1, E_LOC) == s_exp)
        m_exp = m_exp.astype(jnp.float32)
        r_i = lax.broadcasted_iota(jnp.int32, (CHUNK, CHUNK), 0)
        r_j = lax.broadcasted_iota(jnp.int32, (CHUNK, CHUNK), 1)
        lt_low = (r_j < r_i).astype(jnp.float32)
        pos_exp = jnp.dot(lt_low, m_exp,
                          preferred_element_type=jnp.float32)
        u_big = jnp.where(
            jnp.logical_and(
                jnp.logical_and(pos_exp == cap_exp.astype(jnp.float32),
                                m_exp > 0.5),
                s_exp != k),
            1.0, 0.0).astype(jnp.bfloat16)

        for s in range(N_DEV):
            @pl.when(s != k)
            def _():
                recv = pltpu.make_async_remote_copy(
                    src_ref=pack_ref.at[s],
                    dst_ref=recv_ref.at[s],
                    send_sem=send_sems.at[s],
                    recv_sem=recv_sems.at[s],
                    device_id=(s,),
                    device_id_type=pl.DeviceIdType.MESH,
                )
                recv.wait_recv()

        recv_all = recv_ref[:, :, :].reshape(PK, D_OUT)
        out_ref[:, :] = (
            partial_ref[pl.ds(k * CHUNK, CHUNK), :]
            + jnp.dot(u_big, recv_all, preferred_element_type=jnp.float32))

        for d in range(N_DEV):
            @pl.when(d != k)
            def _():
                a2a_copy(d).wait_send()

    return pl.pallas_call(
        body,
        out_shape=jax.ShapeDtypeStruct((CHUNK, D_OUT), jnp.float32),
        in_specs=[
            pl.BlockSpec(memory_space=pltpu.VMEM),
            pl.BlockSpec(memory_space=pltpu.VMEM),
            pl.BlockSpec(memory_space=pltpu.VMEM),
            pl.BlockSpec(memory_space=pltpu.VMEM),
        ],
        out_specs=pl.BlockSpec(memory_space=pltpu.VMEM),
        scratch_shapes=[
            pltpu.VMEM((N_TOK, D_OUT), jnp.float32),
            pltpu.VMEM((N_DEV, CAP, D_OUT), jnp.bfloat16),
            pltpu.VMEM((N_DEV, CAP, D_OUT), jnp.bfloat16),
            pltpu.SemaphoreType.DMA((N_DEV,)),
            pltpu.SemaphoreType.DMA((N_DEV,)),
        ],
        compiler_params=pltpu.CompilerParams(collective_id=0),
    )(x, router_W, route_idx, expert_W)
